# Initial kernel scaffold; baseline (speedup 1.0000x reference)
import jax
import jax.numpy as jnp
from jax.experimental import pallas as pl


def kernel(xyz):
    raise NotImplementedError("write your pallas kernel here")


# speedup vs baseline: 2.4132x; 2.4132x over previous
import jax
import jax.numpy as jnp
from jax.experimental import pallas as pl
from jax.experimental.pallas import tpu as pltpu

_NPOINTS = 512
_LANES = 128


def _fps_pallas(xyz, npoints):
    B, N, _ = xyz.shape
    rows = N // _LANES
    xyzt = jnp.transpose(xyz, (2, 0, 1)).reshape(3, B, rows, _LANES)
    far0 = jax.random.randint(jax.random.key(1), (B,), 0, N).astype(jnp.int32)

    def body(far0_ref, xyzt_ref, out_ref, dist_ref):
        dist_ref[...] = jnp.full((B, rows, _LANES), 1e10, jnp.float32)
        row_iota = jax.lax.broadcasted_iota(jnp.int32, (rows, _LANES), 0)
        lane_iota = jax.lax.broadcasted_iota(jnp.int32, (rows, _LANES), 1)
        flat_iota = row_iota * _LANES + lane_iota
        lane1 = jax.lax.broadcasted_iota(jnp.int32, (1, _LANES), 1)
        b_iota = jax.lax.broadcasted_iota(jnp.int32, (1, B), 1)

        def iter_body(i, farthest):
            rec = jnp.zeros((1, B), jnp.int32)
            for b in range(B):
                rec = jnp.where(b_iota == b, farthest[b], rec)
            out_ref[pl.ds(i, 1), :] = rec

            new_far = []
            for b in range(B):
                f = farthest[b]
                r = f // _LANES
                l = f % _LANES
                xr = xyzt_ref[0, b, pl.ds(r, 1), :]
                yr = xyzt_ref[1, b, pl.ds(r, 1), :]
                zr = xyzt_ref[2, b, pl.ds(r, 1), :]
                sel = lane1 == l
                cx = jnp.sum(jnp.where(sel, xr, 0.0))
                cy = jnp.sum(jnp.where(sel, yr, 0.0))
                cz = jnp.sum(jnp.where(sel, zr, 0.0))
                dx = xyzt_ref[0, b] - cx
                dy = xyzt_ref[1, b] - cy
                dz = xyzt_ref[2, b] - cz
                d = dx * dx + dy * dy + dz * dz
                nd = jnp.minimum(dist_ref[b], d)
                dist_ref[b] = nd
                m = jnp.max(nd)
                f_new = jnp.min(jnp.where(nd == m, flat_iota, N))
                new_far.append(f_new)
            return tuple(new_far)

        jax.lax.fori_loop(
            0, npoints,
            iter_body,
            tuple(far0_ref[b] for b in range(B)),
        )

    out = pl.pallas_call(
        body,
        grid=(),
        in_specs=[
            pl.BlockSpec(memory_space=pltpu.SMEM),
            pl.BlockSpec(memory_space=pltpu.VMEM),
        ],
        out_specs=pl.BlockSpec(memory_space=pltpu.VMEM),
        out_shape=jax.ShapeDtypeStruct((npoints, B), jnp.int32),
        scratch_shapes=[pltpu.VMEM((B, rows, _LANES), jnp.float32)],
    )(far0, xyzt)
    return out.T


def kernel(xyz):
    return _fps_pallas(xyz, _NPOINTS)


# speedup vs baseline: 2.5466x; 1.0553x over previous
import jax
import jax.numpy as jnp
from jax.experimental import pallas as pl
from jax.experimental.pallas import tpu as pltpu

_NPOINTS = 512
_LANES = 128
_CH = 32


def _fps_pallas(xyz, npoints):
    B, N, _ = xyz.shape
    rows = N // _LANES
    ch = min(_CH, rows)
    nchunks = rows // ch
    chunk_elems = ch * _LANES
    xyzt = jnp.transpose(xyz, (2, 0, 1)).reshape(3, B, rows, _LANES)
    far0 = jax.random.randint(jax.random.key(1), (B,), 0, N).astype(jnp.int32)

    def body(far0_ref, xyzt_ref, out_ref, dist_ref):
        dist_ref[...] = jnp.full((B, rows, _LANES), 1e10, jnp.float32)
        lane1 = jax.lax.broadcasted_iota(jnp.int32, (1, _LANES), 1)
        b_iota = jax.lax.broadcasted_iota(jnp.int32, (1, B), 1)
        pos_iota = (
            jax.lax.broadcasted_iota(jnp.int32, (ch, _LANES), 0) * _LANES
            + jax.lax.broadcasted_iota(jnp.int32, (ch, _LANES), 1)
        )

        def iter_body(i, farthest):
            rec = jnp.zeros((1, B), jnp.int32)
            for b in range(B):
                rec = jnp.where(b_iota == b, farthest[b], rec)
            out_ref[pl.ds(i, 1), :] = rec

            cents = []
            for b in range(B):
                f = farthest[b]
                r = f // _LANES
                l = f % _LANES
                sel = lane1 == l
                xr = xyzt_ref[0, b, pl.ds(r, 1), :]
                yr = xyzt_ref[1, b, pl.ds(r, 1), :]
                zr = xyzt_ref[2, b, pl.ds(r, 1), :]
                cx = jnp.sum(jnp.where(sel, xr, 0.0))
                cy = jnp.sum(jnp.where(sel, yr, 0.0))
                cz = jnp.sum(jnp.where(sel, zr, 0.0))
                cents.append((cx, cy, cz))

            new_far = []
            for b in range(B):
                cx, cy, cz = cents[b]

                def chunk_body(k, carry, b=b, cx=cx, cy=cy, cz=cz):
                    rm, ri = carry
                    sl = pl.ds(k * ch, ch)
                    x = xyzt_ref[0, b, sl, :]
                    y = xyzt_ref[1, b, sl, :]
                    z = xyzt_ref[2, b, sl, :]
                    dx = x - cx
                    dy = y - cy
                    dz = z - cz
                    d = dx * dx + dy * dy + dz * dz
                    nd = jnp.minimum(dist_ref[b, sl, :], d)
                    dist_ref[b, sl, :] = nd
                    gt = nd > rm
                    rm = jnp.where(gt, nd, rm)
                    ri = jnp.where(gt, k, ri)
                    return rm, ri

                rm0 = jnp.full((ch, _LANES), -1.0, jnp.float32)
                ri0 = jnp.zeros((ch, _LANES), jnp.int32)
                rm, ri = jax.lax.fori_loop(0, nchunks, chunk_body, (rm0, ri0))
                m = jnp.max(rm)
                cand = ri * chunk_elems + pos_iota
                f_new = jnp.min(jnp.where(rm == m, cand, N))
                new_far.append(f_new)
            return tuple(new_far)

        jax.lax.fori_loop(
            0, npoints,
            iter_body,
            tuple(far0_ref[b] for b in range(B)),
        )

    out = pl.pallas_call(
        body,
        grid=(),
        in_specs=[
            pl.BlockSpec(memory_space=pltpu.SMEM),
            pl.BlockSpec(memory_space=pltpu.VMEM),
        ],
        out_specs=pl.BlockSpec(memory_space=pltpu.VMEM),
        out_shape=jax.ShapeDtypeStruct((npoints, B), jnp.int32),
        scratch_shapes=[pltpu.VMEM((B, rows, _LANES), jnp.float32)],
    )(far0, xyzt)
    return out.T


def kernel(xyz):
    return _fps_pallas(xyz, _NPOINTS)


# speedup vs baseline: 6.0629x; 2.3808x over previous
import jax
import jax.numpy as jnp
from jax.experimental import pallas as pl
from jax.experimental.pallas import tpu as pltpu

_NPOINTS = 512
_LANES = 128
_CH = 32


def _fps_pallas(xyz, npoints):
    B, N, _ = xyz.shape
    rows = N // _LANES
    ch = min(_CH, rows)
    nchunks = rows // ch
    chunk_elems = ch * _LANES
    xyzt = jnp.transpose(xyz, (2, 0, 1)).reshape(3, B, rows, _LANES)
    far0 = jax.random.randint(jax.random.key(1), (B,), 0, N).astype(jnp.int32)

    def body(far0_ref, xyzt_ref, out_ref, dist_ref):
        dist_ref[...] = jnp.full((B, rows, _LANES), 1e10, jnp.float32)
        lane1 = jax.lax.broadcasted_iota(jnp.int32, (1, _LANES), 1)
        b_iota = jax.lax.broadcasted_iota(jnp.int32, (1, B), 1)
        pos_iota = (
            jax.lax.broadcasted_iota(jnp.int32, (ch, _LANES), 0) * _LANES
            + jax.lax.broadcasted_iota(jnp.int32, (ch, _LANES), 1)
        )

        def iter_body(i, farthest):
            rec = jnp.zeros((1, B), jnp.int32)
            for b in range(B):
                rec = jnp.where(b_iota == b, farthest[b], rec)
            out_ref[pl.ds(i, 1), :] = rec

            cents = []
            for b in range(B):
                f = farthest[b]
                r = f // _LANES
                l = f % _LANES
                sel = lane1 == l
                xr = xyzt_ref[0, b, pl.ds(r, 1), :]
                yr = xyzt_ref[1, b, pl.ds(r, 1), :]
                zr = xyzt_ref[2, b, pl.ds(r, 1), :]
                cx = jnp.sum(jnp.where(sel, xr, 0.0))
                cy = jnp.sum(jnp.where(sel, yr, 0.0))
                cz = jnp.sum(jnp.where(sel, zr, 0.0))
                cents.append((cx, cy, cz))

            new_far = []
            for b in range(B):
                cx, cy, cz = cents[b]

                rm = jnp.full((ch, _LANES), -1.0, jnp.float32)
                ri = jnp.zeros((ch, _LANES), jnp.int32)
                for k in range(nchunks):
                    sl = pl.ds(k * ch, ch)
                    x = xyzt_ref[0, b, sl, :]
                    y = xyzt_ref[1, b, sl, :]
                    z = xyzt_ref[2, b, sl, :]
                    dx = x - cx
                    dy = y - cy
                    dz = z - cz
                    d = dx * dx + dy * dy + dz * dz
                    nd = jnp.minimum(dist_ref[b, sl, :], d)
                    dist_ref[b, sl, :] = nd
                    gt = nd > rm
                    rm = jnp.where(gt, nd, rm)
                    ri = jnp.where(gt, k, ri)
                m = jnp.max(rm)
                cand = ri * chunk_elems + pos_iota
                f_new = jnp.min(jnp.where(rm == m, cand, N))
                new_far.append(f_new)
            return tuple(new_far)

        jax.lax.fori_loop(
            0, npoints,
            iter_body,
            tuple(far0_ref[b] for b in range(B)),
        )

    out = pl.pallas_call(
        body,
        grid=(),
        in_specs=[
            pl.BlockSpec(memory_space=pltpu.SMEM),
            pl.BlockSpec(memory_space=pltpu.VMEM),
        ],
        out_specs=pl.BlockSpec(memory_space=pltpu.VMEM),
        out_shape=jax.ShapeDtypeStruct((npoints, B), jnp.int32),
        scratch_shapes=[pltpu.VMEM((B, rows, _LANES), jnp.float32)],
    )(far0, xyzt)
    return out.T


def kernel(xyz):
    return _fps_pallas(xyz, _NPOINTS)
